# baseline (speedup 1.0000x reference)
import functools
import jax
import jax.numpy as jnp
from jax import lax
from jax.experimental import pallas as pl
from jax.experimental.pallas import tpu as pltpu
from jax.experimental.pallas import tpu_sc as plsc

N = 50000
K = 8
C_IN = 256
C_OUT = 128
N_OUT = K * N
EPS = 1e-5

_MPAD = 401408
_B = 512
_NBLK = _MPAD // _B
_NBUK = 56
_PADIDX = 55 << 13
_RSC = 8192
_RZ = 8448
_NPASS = 25
_OUT_PAD = _NPASS * 2 * _RSC
_CH = 128
_NCHT = _MPAD // _CH

_BN = 2000
_NB_MM = N // _BN


def _mm_body(x_ref, w_ref, o_ref):
    o_ref[...] = jnp.dot(x_ref[...], w_ref[0], preferred_element_type=jnp.float32)


def _matmul(x, W):
    return pl.pallas_call(
        _mm_body,
        grid=(_NB_MM, K),
        in_specs=[
            pl.BlockSpec((_BN, C_IN), lambda i, k: (i, 0)),
            pl.BlockSpec((1, C_IN, C_OUT), lambda i, k: (k, 0, 0)),
        ],
        out_specs=pl.BlockSpec((_BN, C_OUT), lambda i, k: (k * _NB_MM + i, 0)),
        out_shape=jax.ShapeDtypeStruct((_MPAD, C_OUT), jnp.float32),
    )(x, W)


def _s1_body(i_ref, cnt_ref, wp_ref):
    v = i_ref[...]
    b = v >> 13
    bi = lax.broadcasted_iota(jnp.int32, (_B, _NBUK), 1)
    oh = (b == bi).astype(jnp.float32)
    cnt_ref[...] = jnp.sum(oh, axis=0, keepdims=True)[None]
    r = lax.broadcasted_iota(jnp.int32, (_B, _B), 0)
    cm = lax.broadcasted_iota(jnp.int32, (_B, _B), 1)
    tril = (cm < r).astype(jnp.float32)
    colcum = jnp.dot(tril, oh, preferred_element_type=jnp.float32,
                     precision=lax.Precision.HIGHEST)
    wp_ref[...] = jnp.sum(colcum * oh, axis=1, keepdims=True)


def _sort_s1(idx2d):
    return pl.pallas_call(
        _s1_body,
        grid=(_NBLK,),
        in_specs=[pl.BlockSpec((_B, 1), lambda i: (i, 0))],
        out_specs=[
            pl.BlockSpec((1, 1, _NBUK), lambda i: (i, 0, 0)),
            pl.BlockSpec((_B, 1), lambda i: (i, 0)),
        ],
        out_shape=[
            jax.ShapeDtypeStruct((_NBLK, 1, _NBUK), jnp.float32),
            jax.ShapeDtypeStruct((_MPAD, 1), jnp.float32),
        ],
    )(idx2d)


def _s2_body(cnt_ref, dbase_ref, bs_ref):
    cnt = cnt_ref[...].reshape(_NBLK, _NBUK)
    r = lax.broadcasted_iota(jnp.int32, (_NBLK, _NBLK), 0)
    cm = lax.broadcasted_iota(jnp.int32, (_NBLK, _NBLK), 1)
    tril = (cm < r).astype(jnp.float32)
    bpref = jnp.dot(tril, cnt, preferred_element_type=jnp.float32,
                    precision=lax.Precision.HIGHEST)
    totals = jnp.sum(cnt, axis=0)
    r2 = lax.broadcasted_iota(jnp.int32, (_NBUK, _NBUK), 0)
    c2 = lax.broadcasted_iota(jnp.int32, (_NBUK, _NBUK), 1)
    tril2 = (c2 < r2).astype(jnp.float32)
    bstart = jnp.dot(tril2, totals[:, None], preferred_element_type=jnp.float32,
                     precision=lax.Precision.HIGHEST)[:, 0]
    dbase_ref[...] = (bpref + bstart[None, :])[:, None, :]
    bs_ref[...] = bstart[None, :].astype(jnp.int32)


def _sort_s2(cnt):
    return pl.pallas_call(
        _s2_body,
        out_shape=[
            jax.ShapeDtypeStruct((_NBLK, 1, _NBUK), jnp.float32),
            jax.ShapeDtypeStruct((1, _NBUK), jnp.int32),
        ],
    )(cnt)


def _s3_body(i_ref, wp_ref, db_ref, d_ref):
    v = i_ref[...]
    b = v >> 13
    bi = lax.broadcasted_iota(jnp.int32, (_B, _NBUK), 1)
    oh = (b == bi).astype(jnp.float32)
    sel = jnp.sum(oh * db_ref[0], axis=1, keepdims=True)
    d_ref[...] = (sel + wp_ref[...]).astype(jnp.int32)


def _sort_s3(idx2d, wpref, dbase):
    return pl.pallas_call(
        _s3_body,
        grid=(_NBLK,),
        in_specs=[
            pl.BlockSpec((_B, 1), lambda i: (i, 0)),
            pl.BlockSpec((_B, 1), lambda i: (i, 0)),
            pl.BlockSpec((1, 1, _NBUK), lambda i: (i, 0, 0)),
        ],
        out_specs=pl.BlockSpec((_B, 1), lambda i: (i, 0)),
        out_shape=jax.ShapeDtypeStruct((_MPAD, 1), jnp.int32),
    )(idx2d, wpref, dbase)


def _perm_body(msg_hbm, dest_hbm, idx2_hbm, smsg_hbm, sidx_hbm,
               dbuf, rb, ib, sem):
    c = lax.axis_index("c")
    s = lax.axis_index("s")
    w = s * 2 + c
    nchk = _MPAD // (32 * _CH)

    def _chunk(t, _):
        ch = w * nchk + t
        pltpu.sync_copy(dest_hbm.at[pl.ds(ch, 1)], dbuf)
        pltpu.sync_copy(msg_hbm.at[pl.ds(ch * _CH, _CH)], rb)
        pltpu.sync_copy(idx2_hbm.at[pl.ds(ch * _CH, _CH)], ib)
        for q in range(8):
            destv = dbuf[0, pl.ds(q * 16, 16)]
            pltpu.sync_copy(rb.at[pl.ds(q * 16, 16)], smsg_hbm.at[destv])
            pltpu.sync_copy(ib.at[pl.ds(q * 16, 16)], sidx_hbm.at[destv])
        return 0
    lax.fori_loop(0, nchk, _chunk, 0)


def _sc_permute(msg, dest2d, idx2):
    mesh = plsc.VectorSubcoreMesh(core_axis_name="c", subcore_axis_name="s")
    f = functools.partial(
        pl.kernel,
        out_type=(
            jax.ShapeDtypeStruct((_MPAD, C_OUT), jnp.float32),
            jax.ShapeDtypeStruct((_MPAD, 128), jnp.int32),
        ),
        mesh=mesh,
        scratch_types=[
            pltpu.VMEM((1, _CH), jnp.int32),
            pltpu.VMEM((_CH, C_OUT), jnp.float32),
            pltpu.VMEM((_CH, 128), jnp.int32),
            pltpu.SemaphoreType.DMA,
        ],
    )(_perm_body)
    return f(msg, dest2d, idx2)


def _acc_body(smsg_hbm, sidx_hbm, bs_hbm, out_hbm,
              rb, ib, zbuf, bsv, shared, sem):
    c = lax.axis_index("c")
    s = lax.axis_index("s")
    lane = lax.iota(jnp.int32, 16)

    def _zb(t, _):
        zbuf[t // 8, pl.ds((t % 8) * 16, 16)] = jnp.zeros((16,), jnp.float32)
        return 0
    lax.fori_loop(0, _CH * 8, _zb, 0)

    pltpu.sync_copy(bs_hbm, bsv)
    bsvec = [bsv[0, pl.ds(16 * g, 16)] for g in range(4)]

    def _bs(i):
        return bsvec[i // 16][i % 16]

    trash = _RSC + s * 16

    for p in range(_NPASS):
        z0 = s * 512
        for t in range(4):
            pltpu.sync_copy(zbuf, shared.at[pl.ds(z0 + t * _CH, _CH)])

        plsc.subcore_barrier()

        lo = (2 * p) * _RSC + c * _RSC
        hi = lo + _RSC
        seg0 = _bs(2 * p) * (1 - c) + _bs(2 * p + 1) * c
        seg1 = _bs(2 * p + 1) * (1 - c) + _bs(2 * p + 2) * c
        c0 = seg0 >> 7
        c1 = (seg1 + _CH - 1) >> 7
        ntile = (c1 - c0 - s + 15) >> 4

        def _chunk(tt, _):
            tc = jnp.minimum(c0 + s + tt * 16, _NCHT - 1)
            pltpu.sync_copy(smsg_hbm.at[pl.ds(tc * _CH, _CH)], rb)
            pltpu.sync_copy(sidx_hbm.at[pl.ds(tc * _CH, _CH)], ib)
            for q in range(8):
                idxv = ib[q * 16, pl.ds(0, 16)] * 0
                for j in range(16):
                    rowv = ib[q * 16 + j, pl.ds(0, 16)]
                    idxv = jnp.where(lane == j, rowv, idxv)
                m = (idxv >= lo) & (idxv < hi)
                locv = jnp.where(m, idxv & (_RSC - 1), trash + lane)
                pltpu.sync_copy(rb.at[pl.ds(q * 16, 16)], shared.at[locv],
                                add=True)
            return 0
        lax.fori_loop(0, ntile, _chunk, 0)

        plsc.subcore_barrier()

        g0 = p * 2 * _RSC + c * _RSC + s * 512
        pltpu.sync_copy(shared.at[pl.ds(s * 512, 512)], out_hbm.at[pl.ds(g0, 512)])

        plsc.subcore_barrier()


def _sc_accumulate(smsg, sidx, bstart):
    mesh = plsc.VectorSubcoreMesh(core_axis_name="c", subcore_axis_name="s")
    f = functools.partial(
        pl.kernel,
        out_type=jax.ShapeDtypeStruct((_OUT_PAD, C_OUT), jnp.float32),
        mesh=mesh,
        scratch_types=[
            pltpu.VMEM((_CH, C_OUT), jnp.float32),
            pltpu.VMEM((_CH, 128), jnp.int32),
            pltpu.VMEM((_CH, C_OUT), jnp.float32),
            pltpu.VMEM((1, 64), jnp.int32),
            pltpu.VMEM_SHARED((_RZ, C_OUT), jnp.float32),
            pltpu.SemaphoreType.DMA,
        ],
    )(_acc_body)
    return f(smsg, sidx, bstart)


_BS = 4000
_NSB = N_OUT // _BS


def _stats_body(o_ref, s_ref, acc):
    @pl.when(pl.program_id(0) == 0)
    def _():
        acc[...] = jnp.zeros_like(acc)

    x = o_ref[...]
    acc[0, :] += jnp.sum(x, axis=0)
    acc[1, :] += jnp.sum(x * x, axis=0)

    @pl.when(pl.program_id(0) == _NSB - 1)
    def _():
        s_ref[...] = acc[...]


def _stats(out):
    return pl.pallas_call(
        _stats_body,
        grid=(_NSB,),
        in_specs=[pl.BlockSpec((_BS, C_OUT), lambda i: (i, 0))],
        out_specs=pl.BlockSpec((2, C_OUT), lambda i: (0, 0)),
        out_shape=jax.ShapeDtypeStruct((2, C_OUT), jnp.float32),
        scratch_shapes=[pltpu.VMEM((2, C_OUT), jnp.float32)],
    )(out)


def _norm_body(o_ref, s_ref, g_ref, b_ref, y_ref):
    ssum = s_ref[0, :]
    ssq = s_ref[1, :]
    inv_n = jnp.float32(1.0 / N_OUT)
    mean = ssum * inv_n
    var = ssq * inv_n - mean * mean
    scale = g_ref[0] * jax.lax.rsqrt(var + EPS)
    shift = b_ref[0] - mean * scale
    h = o_ref[...] * scale[None, :] + shift[None, :]
    y_ref[...] = h * 0.5 * (1.0 + jax.lax.erf(h * jnp.float32(0.7071067811865476)))


def _normalize(out, stats, gamma, beta):
    return pl.pallas_call(
        _norm_body,
        grid=(_NSB,),
        in_specs=[
            pl.BlockSpec((_BS, C_OUT), lambda i: (i, 0)),
            pl.BlockSpec((2, C_OUT), lambda i: (0, 0)),
            pl.BlockSpec((1, C_OUT), lambda i: (0, 0)),
            pl.BlockSpec((1, C_OUT), lambda i: (0, 0)),
        ],
        out_specs=pl.BlockSpec((_BS, C_OUT), lambda i: (i, 0)),
        out_shape=jax.ShapeDtypeStruct((N_OUT, C_OUT), jnp.float32),
    )(out, stats, gamma, beta)


def kernel(x, W, gamma, beta, out_map):
    msg = _matmul(x, W)
    idx_pad = jnp.concatenate(
        [out_map.reshape(-1),
         jnp.full((_MPAD - N_OUT,), _PADIDX, jnp.int32)])
    idx2d = idx_pad.reshape(_MPAD, 1)
    cnt, wpref = _sort_s1(idx2d)
    dbase, bstart = _sort_s2(cnt)
    dest = _sort_s3(idx2d, wpref, dbase)
    dest2d = dest.reshape(_NCHT, _CH)
    idx2 = jnp.broadcast_to(idx_pad[:, None], (_MPAD, 128))
    smsg, sidx = _sc_permute(msg, dest2d, idx2)
    bs64 = jnp.zeros((1, 64), jnp.int32).at[0, :_NBUK].set(bstart[0])
    out = _sc_accumulate(smsg, sidx, bs64)
    st = _stats(out)
    return _normalize(out, st, gamma.reshape(1, C_OUT), beta.reshape(1, C_OUT))


# speedup vs baseline: 1.0264x; 1.0264x over previous
import functools
import jax
import jax.numpy as jnp
from jax import lax
from jax.experimental import pallas as pl
from jax.experimental.pallas import tpu as pltpu
from jax.experimental.pallas import tpu_sc as plsc

N = 50000
K = 8
C_IN = 256
C_OUT = 128
N_OUT = K * N
EPS = 1e-5

_MPAD = 401408
_B = 512
_NBLK = _MPAD // _B
_NBUK = 56
_PADIDX = 55 << 13
_RSC = 8192
_RZ = 8448
_NPASS = 25
_OUT_PAD = _NPASS * 2 * _RSC
_CH = 128
_NCHT = _MPAD // _CH

_BN = 2000
_NB_MM = N // _BN


def _mm_body(x_ref, w_ref, o_ref):
    o_ref[...] = jnp.dot(x_ref[...].astype(jnp.bfloat16),
                         w_ref[0].astype(jnp.bfloat16),
                         preferred_element_type=jnp.float32)


def _matmul(x, W):
    return pl.pallas_call(
        _mm_body,
        grid=(_NB_MM, K),
        in_specs=[
            pl.BlockSpec((_BN, C_IN), lambda i, k: (i, 0)),
            pl.BlockSpec((1, C_IN, C_OUT), lambda i, k: (k, 0, 0)),
        ],
        out_specs=pl.BlockSpec((_BN, C_OUT), lambda i, k: (k * _NB_MM + i, 0)),
        out_shape=jax.ShapeDtypeStruct((_MPAD, C_OUT), jnp.float32),
    )(x, W)


def _s1_body(i_ref, cnt_ref, wp_ref):
    v = i_ref[...]
    b = v >> 13
    bi = lax.broadcasted_iota(jnp.int32, (_B, _NBUK), 1)
    oh = (b == bi).astype(jnp.float32)
    cnt_ref[...] = jnp.sum(oh, axis=0, keepdims=True)[None]
    r = lax.broadcasted_iota(jnp.int32, (_B, _B), 0)
    cm = lax.broadcasted_iota(jnp.int32, (_B, _B), 1)
    tril = (cm < r).astype(jnp.float32)
    colcum = jnp.dot(tril, oh, preferred_element_type=jnp.float32,
                     precision=lax.Precision.HIGHEST)
    wp_ref[...] = jnp.sum(colcum * oh, axis=1, keepdims=True)


def _sort_s1(idx2d):
    return pl.pallas_call(
        _s1_body,
        grid=(_NBLK,),
        in_specs=[pl.BlockSpec((_B, 1), lambda i: (i, 0))],
        out_specs=[
            pl.BlockSpec((1, 1, _NBUK), lambda i: (i, 0, 0)),
            pl.BlockSpec((_B, 1), lambda i: (i, 0)),
        ],
        out_shape=[
            jax.ShapeDtypeStruct((_NBLK, 1, _NBUK), jnp.float32),
            jax.ShapeDtypeStruct((_MPAD, 1), jnp.float32),
        ],
    )(idx2d)


def _s2_body(cnt_ref, dbase_ref, bs_ref):
    cnt = cnt_ref[...].reshape(_NBLK, _NBUK)
    r = lax.broadcasted_iota(jnp.int32, (_NBLK, _NBLK), 0)
    cm = lax.broadcasted_iota(jnp.int32, (_NBLK, _NBLK), 1)
    tril = (cm < r).astype(jnp.float32)
    bpref = jnp.dot(tril, cnt, preferred_element_type=jnp.float32,
                    precision=lax.Precision.HIGHEST)
    totals = jnp.sum(cnt, axis=0)
    r2 = lax.broadcasted_iota(jnp.int32, (_NBUK, _NBUK), 0)
    c2 = lax.broadcasted_iota(jnp.int32, (_NBUK, _NBUK), 1)
    tril2 = (c2 < r2).astype(jnp.float32)
    bstart = jnp.dot(tril2, totals[:, None], preferred_element_type=jnp.float32,
                     precision=lax.Precision.HIGHEST)[:, 0]
    dbase_ref[...] = (bpref + bstart[None, :])[:, None, :]
    bs_ref[...] = bstart[None, :].astype(jnp.int32)


def _sort_s2(cnt):
    return pl.pallas_call(
        _s2_body,
        out_shape=[
            jax.ShapeDtypeStruct((_NBLK, 1, _NBUK), jnp.float32),
            jax.ShapeDtypeStruct((1, _NBUK), jnp.int32),
        ],
    )(cnt)


def _s3_body(i_ref, wp_ref, db_ref, d_ref):
    v = i_ref[...]
    b = v >> 13
    bi = lax.broadcasted_iota(jnp.int32, (_B, _NBUK), 1)
    oh = (b == bi).astype(jnp.float32)
    sel = jnp.sum(oh * db_ref[0], axis=1, keepdims=True)
    d_ref[...] = (sel + wp_ref[...]).astype(jnp.int32)


def _sort_s3(idx2d, wpref, dbase):
    return pl.pallas_call(
        _s3_body,
        grid=(_NBLK,),
        in_specs=[
            pl.BlockSpec((_B, 1), lambda i: (i, 0)),
            pl.BlockSpec((_B, 1), lambda i: (i, 0)),
            pl.BlockSpec((1, 1, _NBUK), lambda i: (i, 0, 0)),
        ],
        out_specs=pl.BlockSpec((_B, 1), lambda i: (i, 0)),
        out_shape=jax.ShapeDtypeStruct((_MPAD, 1), jnp.int32),
    )(idx2d, wpref, dbase)


def _perm_body(msg_hbm, dest_hbm, idx2_hbm, smsg_hbm, sidx_hbm,
               dbuf, rb, ib, sem):
    c = lax.axis_index("c")
    s = lax.axis_index("s")
    w = s * 2 + c
    nchk = _MPAD // (32 * _CH)

    def _chunk(t, _):
        ch = w * nchk + t
        pltpu.sync_copy(dest_hbm.at[pl.ds(ch, 1)], dbuf)
        pltpu.sync_copy(msg_hbm.at[pl.ds(ch * _CH, _CH)], rb)
        pltpu.sync_copy(idx2_hbm.at[pl.ds(ch * _CH, _CH)], ib)
        pltpu.sync_copy(rb, smsg_hbm.at[dbuf.at[0]])
        pltpu.sync_copy(ib, sidx_hbm.at[dbuf.at[0]])
        return 0
    lax.fori_loop(0, nchk, _chunk, 0)


def _sc_permute(msg, dest2d, idx2):
    mesh = plsc.VectorSubcoreMesh(core_axis_name="c", subcore_axis_name="s")
    f = functools.partial(
        pl.kernel,
        out_type=(
            jax.ShapeDtypeStruct((_MPAD, C_OUT), jnp.float32),
            jax.ShapeDtypeStruct((_MPAD, 128), jnp.int32),
        ),
        mesh=mesh,
        scratch_types=[
            pltpu.VMEM((1, _CH), jnp.int32),
            pltpu.VMEM((_CH, C_OUT), jnp.float32),
            pltpu.VMEM((_CH, 128), jnp.int32),
            pltpu.SemaphoreType.DMA,
        ],
    )(_perm_body)
    return f(msg, dest2d, idx2)


def _acc_body(smsg_hbm, sidx_hbm, bs_hbm, out_hbm,
              rb, ib, zbuf, bsv, shared, sem):
    c = lax.axis_index("c")
    s = lax.axis_index("s")
    lane = lax.iota(jnp.int32, 16)

    def _zb(t, _):
        zbuf[t // 8, pl.ds((t % 8) * 16, 16)] = jnp.zeros((16,), jnp.float32)
        return 0
    lax.fori_loop(0, _CH * 8, _zb, 0)

    pltpu.sync_copy(bs_hbm, bsv)
    bsvec = [bsv[0, pl.ds(16 * g, 16)] for g in range(4)]

    def _bs(i):
        return bsvec[i // 16][i % 16]

    trash = _RSC + s * 16

    for p in range(_NPASS):
        z0 = s * 512
        for t in range(4):
            pltpu.sync_copy(zbuf, shared.at[pl.ds(z0 + t * _CH, _CH)])

        plsc.subcore_barrier()

        lo = (2 * p) * _RSC + c * _RSC
        hi = lo + _RSC
        seg0 = _bs(2 * p) * (1 - c) + _bs(2 * p + 1) * c
        seg1 = _bs(2 * p + 1) * (1 - c) + _bs(2 * p + 2) * c
        c0 = seg0 >> 7
        c1 = (seg1 + _CH - 1) >> 7
        ntile = (c1 - c0 - s + 15) >> 4

        def _chunk(tt, _):
            tc = jnp.minimum(c0 + s + tt * 16, _NCHT - 1)
            pltpu.sync_copy(smsg_hbm.at[pl.ds(tc * _CH, _CH)], rb)
            pltpu.sync_copy(sidx_hbm.at[pl.ds(tc * _CH, _CH)], ib)
            for q in range(8):
                idxv = ib[q * 16, pl.ds(0, 16)] * 0
                for j in range(16):
                    rowv = ib[q * 16 + j, pl.ds(0, 16)]
                    idxv = jnp.where(lane == j, rowv, idxv)
                m = (idxv >= lo) & (idxv < hi)
                locv = jnp.where(m, idxv & (_RSC - 1), trash + lane)
                pltpu.sync_copy(rb.at[pl.ds(q * 16, 16)], shared.at[locv],
                                add=True)
            return 0
        lax.fori_loop(0, ntile, _chunk, 0)

        plsc.subcore_barrier()

        g0 = p * 2 * _RSC + c * _RSC + s * 512
        pltpu.sync_copy(shared.at[pl.ds(s * 512, 512)], out_hbm.at[pl.ds(g0, 512)])

        plsc.subcore_barrier()


def _sc_accumulate(smsg, sidx, bstart):
    mesh = plsc.VectorSubcoreMesh(core_axis_name="c", subcore_axis_name="s")
    f = functools.partial(
        pl.kernel,
        out_type=jax.ShapeDtypeStruct((_OUT_PAD, C_OUT), jnp.float32),
        mesh=mesh,
        scratch_types=[
            pltpu.VMEM((_CH, C_OUT), jnp.float32),
            pltpu.VMEM((_CH, 128), jnp.int32),
            pltpu.VMEM((_CH, C_OUT), jnp.float32),
            pltpu.VMEM((1, 64), jnp.int32),
            pltpu.VMEM_SHARED((_RZ, C_OUT), jnp.float32),
            pltpu.SemaphoreType.DMA,
        ],
    )(_acc_body)
    return f(smsg, sidx, bstart)


_BS = 4000
_NSB = N_OUT // _BS


def _stats_body(o_ref, s_ref, acc):
    @pl.when(pl.program_id(0) == 0)
    def _():
        acc[...] = jnp.zeros_like(acc)

    x = o_ref[...]
    acc[0, :] += jnp.sum(x, axis=0)
    acc[1, :] += jnp.sum(x * x, axis=0)

    @pl.when(pl.program_id(0) == _NSB - 1)
    def _():
        s_ref[...] = acc[...]


def _stats(out):
    return pl.pallas_call(
        _stats_body,
        grid=(_NSB,),
        in_specs=[pl.BlockSpec((_BS, C_OUT), lambda i: (i, 0))],
        out_specs=pl.BlockSpec((2, C_OUT), lambda i: (0, 0)),
        out_shape=jax.ShapeDtypeStruct((2, C_OUT), jnp.float32),
        scratch_shapes=[pltpu.VMEM((2, C_OUT), jnp.float32)],
    )(out)


def _norm_body(o_ref, s_ref, g_ref, b_ref, y_ref):
    ssum = s_ref[0, :]
    ssq = s_ref[1, :]
    inv_n = jnp.float32(1.0 / N_OUT)
    mean = ssum * inv_n
    var = ssq * inv_n - mean * mean
    scale = g_ref[0] * jax.lax.rsqrt(var + EPS)
    shift = b_ref[0] - mean * scale
    h = o_ref[...] * scale[None, :] + shift[None, :]
    y_ref[...] = h * 0.5 * (1.0 + jax.lax.erf(h * jnp.float32(0.7071067811865476)))


def _normalize(out, stats, gamma, beta):
    return pl.pallas_call(
        _norm_body,
        grid=(_NSB,),
        in_specs=[
            pl.BlockSpec((_BS, C_OUT), lambda i: (i, 0)),
            pl.BlockSpec((2, C_OUT), lambda i: (0, 0)),
            pl.BlockSpec((1, C_OUT), lambda i: (0, 0)),
            pl.BlockSpec((1, C_OUT), lambda i: (0, 0)),
        ],
        out_specs=pl.BlockSpec((_BS, C_OUT), lambda i: (i, 0)),
        out_shape=jax.ShapeDtypeStruct((N_OUT, C_OUT), jnp.float32),
    )(out, stats, gamma, beta)


def kernel(x, W, gamma, beta, out_map):
    msg = _matmul(x, W)
    idx_pad = jnp.concatenate(
        [out_map.reshape(-1),
         jnp.full((_MPAD - N_OUT,), _PADIDX, jnp.int32)])
    idx2d = idx_pad.reshape(_MPAD, 1)
    cnt, wpref = _sort_s1(idx2d)
    dbase, bstart = _sort_s2(cnt)
    dest = _sort_s3(idx2d, wpref, dbase)
    dest2d = dest.reshape(_NCHT, _CH)
    idx2 = jnp.broadcast_to(idx_pad[:, None], (_MPAD, 128))
    smsg, sidx = _sc_permute(msg, dest2d, idx2)
    bs64 = jnp.zeros((1, 64), jnp.int32).at[0, :_NBUK].set(bstart[0])
    out = _sc_accumulate(smsg, sidx, bs64)
    st = _stats(out)
    return _normalize(out, st, gamma.reshape(1, C_OUT), beta.reshape(1, C_OUT))
